# Initial kernel scaffold; baseline (speedup 1.0000x reference)
import jax
import jax.numpy as jnp
from jax.experimental import pallas as pl


def kernel(user, item, table):
    raise NotImplementedError("write your pallas kernel here")


# speedup vs baseline: 3.3420x; 3.3420x over previous
import jax
import jax.numpy as jnp
from jax import lax
from jax.experimental import pallas as pl
from jax.experimental.pallas import tpu as pltpu
from jax.experimental.pallas import tpu_sc as plsc

_VOCAB_PAD = 1024
_BATCH = 16384
_NW = 32
_ROWS = 4
_ROW_W = 128


def _lookup_kernel(table_hbm, item_hbm, out_hbm, idx_v, out_v, sem):
    wid = lax.axis_index("s") * 2 + lax.axis_index("c")
    pltpu.sync_copy(item_hbm.at[wid], idx_v)
    copies = [
        pltpu.async_copy(table_hbm.at[idx_v.at[j]], out_v.at[j], sem)
        for j in range(_ROWS)
    ]
    for c in copies:
        c.wait()
    pltpu.sync_copy(out_v, out_hbm.at[wid])


@jax.jit
def kernel(user, item, table):
    del user
    flat = jnp.zeros((_VOCAB_PAD,), jnp.float32).at[: table.shape[0]].set(
        table[:, 0]
    )
    item3 = item.reshape(_NW, _ROWS, _ROW_W)
    mesh = plsc.VectorSubcoreMesh(core_axis_name="c", subcore_axis_name="s")
    run = pl.kernel(
        _lookup_kernel,
        mesh=mesh,
        out_type=jax.ShapeDtypeStruct((_NW, _ROWS, _ROW_W), jnp.float32),
        scratch_types=[
            pltpu.VMEM((_ROWS, _ROW_W), jnp.int32),
            pltpu.VMEM((_ROWS, _ROW_W), jnp.float32),
            pltpu.SemaphoreType.DMA,
        ],
    )
    return run(flat, item3).reshape(_BATCH)
